# baseline (speedup 1.0000x reference)
import functools

import jax
import jax.numpy as jnp
from jax import lax
from jax.experimental import pallas as pl
from jax.experimental.pallas import tpu as pltpu
from jax.experimental.pallas import tpu_sc as plsc

DIM = 64
CHUNK = 128
LANES = 16


@functools.lru_cache(maxsize=None)
def _build(batch: int, num_e: int, num_r: int):
    info = plsc.get_sparse_core_info()
    nc, ns = info.num_cores, info.num_subcores
    nw = nc * ns
    num_chunks = batch // CHUNK
    assert num_chunks % nw == 0 and batch % CHUNK == 0
    chunks_per_w = num_chunks // nw

    mesh = plsc.VectorSubcoreMesh(core_axis_name="c", subcore_axis_name="s")

    @functools.partial(
        pl.kernel,
        mesh=mesh,
        compiler_params=pltpu.CompilerParams(
            needs_layout_passes=False, use_tc_tiling_on_sc=False
        ),
        out_type=(
            jax.ShapeDtypeStruct((batch,), jnp.float32),
            jax.ShapeDtypeStruct((batch,), jnp.float32),
        ),
        scratch_types=[
            pltpu.VMEM((CHUNK,), jnp.int32),
            pltpu.VMEM((CHUNK,), jnp.int32),
            pltpu.VMEM((CHUNK,), jnp.int32),
            pltpu.VMEM((CHUNK, DIM), jnp.float32),
            pltpu.VMEM((CHUNK, DIM), jnp.float32),
            pltpu.VMEM((CHUNK, DIM), jnp.float32),
            pltpu.VMEM((CHUNK,), jnp.float32),
            pltpu.SemaphoreType.DMA,
        ],
    )
    def scored(idx_hbm, e_hbm, r_hbm, out0_hbm, out1_hbm,
               hi_v, ri_v, ti_v, h_v, r_v, t_v, out_v, sem):
        wid = lax.axis_index("s") * nc + lax.axis_index("c")
        lane = lax.iota(jnp.int32, LANES)

        def do_chunk(part, chunk):
            base = part * 3
            pltpu.sync_copy(idx_hbm.at[base + 0, chunk], hi_v)
            pltpu.sync_copy(idx_hbm.at[base + 1, chunk], ri_v)
            pltpu.sync_copy(idx_hbm.at[base + 2, chunk], ti_v)
            ch = pltpu.async_copy(e_hbm.at[hi_v], h_v, sem)
            cr = pltpu.async_copy(r_hbm.at[ri_v], r_v, sem)
            ct = pltpu.async_copy(e_hbm.at[ti_v], t_v, sem)
            ch.wait()
            cr.wait()
            ct.wait()

            def group(g, _):
                res = jnp.zeros((LANES,), jnp.float32)
                for j in range(LANES):
                    i = g * LANES + j
                    acc = jnp.zeros((LANES,), jnp.float32)
                    for k in range(DIM // LANES):
                        sl = pl.ds(k * LANES, LANES)
                        e = h_v[i, sl] + r_v[i, sl] - t_v[i, sl]
                        acc = acc + e * e
                    s = jnp.sum(acc)
                    res = jnp.where(lane == j, s, res)
                out_v[pl.ds(g * LANES, LANES)] = res
                return 0

            lax.fori_loop(0, CHUNK // LANES, group, 0)
            out_ref = out0_hbm if part == 0 else out1_hbm
            pltpu.sync_copy(out_v, out_ref.at[pl.ds(chunk * CHUNK, CHUNK)])

        def per_worker(cc, _):
            chunk = wid * chunks_per_w + cc
            do_chunk(0, chunk)
            do_chunk(1, chunk)
            return 0

        lax.fori_loop(0, chunks_per_w, per_worker, 0)

    return scored


def kernel(X, Xc, E_weight, R_weight):
    batch = X.shape[0]
    idx = jnp.stack(
        [X[:, 0], X[:, 1], X[:, 2], Xc[:, 0], Xc[:, 1], Xc[:, 2]]
    ).astype(jnp.int32).reshape(6, batch // CHUNK, CHUNK)
    scored = _build(batch, E_weight.shape[0], R_weight.shape[0])
    d0, d1 = scored(idx, E_weight, R_weight)
    return (d0, d1)


# speedup vs baseline: 1.4884x; 1.4884x over previous
import functools

import jax
import jax.numpy as jnp
from jax import lax
from jax.experimental import pallas as pl
from jax.experimental.pallas import tpu as pltpu
from jax.experimental.pallas import tpu_sc as plsc

DIM = 64
LANES = 16


@functools.lru_cache(maxsize=None)
def _build(batch: int, num_e: int, num_r: int):
    info = plsc.get_sparse_core_info()
    nc, ns = info.num_cores, info.num_subcores
    nw = nc * ns
    per_w = batch // nw
    n_groups = per_w // LANES
    assert batch % (nw * LANES) == 0

    mesh = plsc.VectorSubcoreMesh(core_axis_name="c", subcore_axis_name="s")

    @functools.partial(
        pl.kernel,
        mesh=mesh,
        compiler_params=pltpu.CompilerParams(
            needs_layout_passes=False, use_tc_tiling_on_sc=True
        ),
        out_type=(
            jax.ShapeDtypeStruct((batch,), jnp.float32),
            jax.ShapeDtypeStruct((batch,), jnp.float32),
        ),
        scratch_types=[
            pltpu.VMEM((6, 512), jnp.int32),
            pltpu.VMEM((256, DIM), jnp.float32),
            pltpu.VMEM((256, DIM), jnp.float32),
            pltpu.VMEM((256, DIM), jnp.float32),
            pltpu.VMEM((256,), jnp.float32),
            pltpu.SemaphoreType.DMA,
        ],
    )
    def scored(idx_hbm, e_hbm, r_hbm, out0_hbm, out1_hbm,
               idx_v, h_v, r_v, t_v, out_v, sem):
        wid = lax.axis_index("s") * nc + lax.axis_index("c")
        base = wid * per_w

        for comp in range(6):
            pltpu.sync_copy(idx_hbm.at[comp, pl.ds(base, per_w)],
                            idx_v.at[comp])

        def do_chunk(part, half):
            off = half * 256

            def group(g, _):
                gsl = pl.ds(g * LANES, LANES)
                isl = pl.ds(off + g * LANES, LANES)
                hv = idx_v[part * 3 + 0, isl]
                rv = idx_v[part * 3 + 1, isl]
                tv = idx_v[part * 3 + 2, isl]
                copies = []
                for j in range(LANES):
                    c = g * LANES + j
                    copies.append(pltpu.async_copy(
                        e_hbm.at[hv[j]], h_v.at[c], sem))
                    copies.append(pltpu.async_copy(
                        r_hbm.at[rv[j]], r_v.at[c], sem))
                    copies.append(pltpu.async_copy(
                        e_hbm.at[tv[j]], t_v.at[c], sem))
                for cp in copies:
                    cp.wait()

                lane = lax.iota(jnp.int32, LANES)
                res = jnp.zeros((LANES,), jnp.float32)
                for j in range(LANES):
                    c = g * LANES + j
                    acc = jnp.zeros((LANES,), jnp.float32)
                    for k in range(DIM // LANES):
                        sl = pl.ds(k * LANES, LANES)
                        e = h_v[c, sl] + r_v[c, sl] - t_v[c, sl]
                        acc = acc + e * e
                    res = jnp.where(lane == j, jnp.sum(acc), res)
                out_v[gsl] = res
                return 0

            lax.fori_loop(0, 256 // LANES, group, 0)
            out_ref = out0_hbm if part == 0 else out1_hbm
            pltpu.sync_copy(out_v, out_ref.at[pl.ds(base + off, 256)])

        for part in range(2):
            for half in range(2):
                do_chunk(part, half)

    return scored


def kernel(X, Xc, E_weight, R_weight):
    batch = X.shape[0]
    idx = jnp.stack(
        [X[:, 0], X[:, 1], X[:, 2], Xc[:, 0], Xc[:, 1], Xc[:, 2]]
    ).astype(jnp.int32)
    d0, d1 = _build(batch, E_weight.shape[0], R_weight.shape[0])(
        idx, E_weight, R_weight)
    return (d0, d1)
